# baseline (speedup 1.0000x reference)
import functools

import jax
import jax.numpy as jnp
from jax import lax
from jax.experimental import pallas as pl
from jax.experimental.pallas import tpu as pltpu
from jax.experimental.pallas import tpu_sc as plsc

N = 10000
E = 160000
IN = 256
HID = 512
OUT = 256
G = 64

NS = 16
RCH = 632
RLAST = N - 15 * RCH
EB = 128
ER = E // EB
TR = 80
AR = 64
HB = EB // 2
DR = 40

R = 1000
NBLK = N // R
HF = IN // 2

_f32 = jnp.float32


def _sc_mesh():
    return plsc.VectorSubcoreMesh(core_axis_name="c", subcore_axis_name="s")


def _deg_sc(ones_n, dst2):
    @functools.partial(
        pl.kernel,
        out_type=[jax.ShapeDtypeStruct((N,), _f32),
                  jax.ShapeDtypeStruct((N,), _f32)],
        mesh=_sc_mesh(),
        scratch_types=[
            pltpu.VMEM((DR, EB), jnp.int32),
            pltpu.VMEM((EB,), _f32),
            pltpu.VMEM((1000,), _f32),
            pltpu.VMEM_SHARED((N,), _f32),
        ],
    )
    def k(ones_h, dst_h, deg0_h, deg1_h, dst_v, one_v, tmp_v, deg_s):
        cid = lax.axis_index("c")
        sid = lax.axis_index("s")
        wid = cid * NS + sid

        @pl.when(wid < 31)
        def _():
            pltpu.sync_copy(dst_h.at[pl.ds(wid * DR, DR)], dst_v)

        @pl.when(wid == 31)
        def _():
            pltpu.sync_copy(dst_h.at[pl.ds(31 * DR, 10)],
                            dst_v.at[pl.ds(0, 10)])

        pltpu.sync_copy(ones_h.at[pl.ds(0, EB)], one_v)

        @pl.when(sid < 10)
        def _():
            pltpu.sync_copy(ones_h.at[pl.ds(sid * 1000, 1000)], tmp_v)
            pltpu.sync_copy(tmp_v, deg_s.at[pl.ds(sid * 1000, 1000)])

        plsc.subcore_barrier()
        nrow = jnp.where(wid < 31, DR, 10)

        def body(j, carry):
            pltpu.sync_copy(one_v, deg_s.at[dst_v.at[j]], add=True)
            return carry

        lax.fori_loop(0, nrow, body, 0)
        plsc.subcore_barrier()

        @pl.when(sid < 10)
        def _():
            pltpu.sync_copy(deg_s.at[pl.ds(sid * 1000, 1000)], tmp_v)

            @pl.when(cid == 0)
            def _():
                pltpu.sync_copy(tmp_v, deg0_h.at[pl.ds(sid * 1000, 1000)])

            @pl.when(cid == 1)
            def _():
                pltpu.sync_copy(tmp_v, deg1_h.at[pl.ds(sid * 1000, 1000)])

    return k(ones_n, dst2)


def _agg_sc(xs0, xs1, src2, dst2):
    @functools.partial(
        pl.kernel,
        out_type=[jax.ShapeDtypeStruct((N, HF), _f32),
                  jax.ShapeDtypeStruct((N, HF), _f32)],
        mesh=_sc_mesh(),
        scratch_types=[
            pltpu.VMEM((AR, EB), jnp.int32),
            pltpu.VMEM((AR, EB), jnp.int32),
            pltpu.VMEM((HB, HF), _f32),
            pltpu.VMEM((HB, HF), _f32),
            pltpu.VMEM((HB, HF), _f32),
            pltpu.VMEM((HB, HF), _f32),
            pltpu.VMEM_SHARED((N, HF), _f32),
            pltpu.SemaphoreType.DMA,
            pltpu.SemaphoreType.DMA,
            pltpu.SemaphoreType.DMA,
            pltpu.SemaphoreType.DMA,
        ],
    )
    def k(xs0_h, xs1_h, src_h, dst_h, out0_h, out1_h,
          src_v, dst_v, rows_a, rows_b, rows_c, rows_d, acc_s,
          sem_g0, sem_g1, sem_s0, sem_s1):
        cid = lax.axis_index("c")
        sid = lax.axis_index("s")
        r0 = sid * RCH

        def run(xs_h, out_h):
            @pl.when(sid < 15)
            def _():
                pltpu.sync_copy(xs_h.at[pl.ds(r0, RCH)],
                                acc_s.at[pl.ds(r0, RCH)])

            @pl.when(sid == 15)
            def _():
                pltpu.sync_copy(xs_h.at[pl.ds(15 * RCH, RLAST)],
                                acc_s.at[pl.ds(15 * RCH, RLAST)])

            plsc.subcore_barrier()

            def gather(row, half, buf, sem):
                pltpu.async_copy(
                    xs_h.at[src_v.at[row, pl.ds(half * HB, HB)]], buf, sem)

            def gather_wait(row, half, buf, sem):
                pltpu.make_async_copy(
                    xs_h.at[src_v.at[row, pl.ds(half * HB, HB)]], buf,
                    sem).wait()

            def scat(row, half, buf, sem):
                pltpu.async_copy(
                    buf, acc_s.at[dst_v.at[row, pl.ds(half * HB, HB)]],
                    sem, add=True)

            def scat_wait(row, half, buf, sem):
                pltpu.make_async_copy(
                    buf, acc_s.at[dst_v.at[row, pl.ds(half * HB, HB)]],
                    sem).wait()

            def do_edges(npair):
                @pl.when(npair > 0)
                def _():
                    gather(0, 0, rows_a, sem_g0)
                    gather(0, 1, rows_b, sem_g0)
                    gather(1, 0, rows_c, sem_g1)
                    gather(1, 1, rows_d, sem_g1)

                    def body(t, carry):
                        r_e = 2 * t
                        r_o = 2 * t + 1
                        gather_wait(r_e, 0, rows_a, sem_g0)
                        gather_wait(r_e, 1, rows_b, sem_g0)
                        scat(r_e, 0, rows_a, sem_s0)
                        scat(r_e, 1, rows_b, sem_s0)
                        gather_wait(r_o, 0, rows_c, sem_g1)
                        gather_wait(r_o, 1, rows_d, sem_g1)
                        scat(r_o, 0, rows_c, sem_s1)
                        scat(r_o, 1, rows_d, sem_s1)
                        scat_wait(r_e, 0, rows_a, sem_s0)
                        scat_wait(r_e, 1, rows_b, sem_s0)

                        @pl.when(t + 1 < npair)
                        def _():
                            gather(r_e + 2, 0, rows_a, sem_g0)
                            gather(r_e + 2, 1, rows_b, sem_g0)

                        scat_wait(r_o, 0, rows_c, sem_s1)
                        scat_wait(r_o, 1, rows_d, sem_s1)

                        @pl.when(t + 1 < npair)
                        def _():
                            gather(r_o + 2, 0, rows_c, sem_g1)
                            gather(r_o + 2, 1, rows_d, sem_g1)

                        return carry

                    lax.fori_loop(0, npair, body, 0)

            @pl.when(sid < 15)
            def _():
                pltpu.sync_copy(src_h.at[pl.ds(sid * TR, AR)], src_v)
                pltpu.sync_copy(dst_h.at[pl.ds(sid * TR, AR)], dst_v)

            @pl.when(sid == 15)
            def _():
                pltpu.sync_copy(src_h.at[pl.ds(15 * TR, 48)],
                                src_v.at[pl.ds(0, 48)])
                pltpu.sync_copy(src_h.at[pl.ds(15 * TR + 48, 2)],
                                src_v.at[pl.ds(48, 2)])
                pltpu.sync_copy(dst_h.at[pl.ds(15 * TR, 48)],
                                dst_v.at[pl.ds(0, 48)])
                pltpu.sync_copy(dst_h.at[pl.ds(15 * TR + 48, 2)],
                                dst_v.at[pl.ds(48, 2)])

            do_edges(jnp.where(sid < 15, AR // 2, 25))

            @pl.when(sid < 15)
            def _():
                pltpu.sync_copy(src_h.at[pl.ds(sid * TR + AR, TR - AR)],
                                src_v.at[pl.ds(0, TR - AR)])
                pltpu.sync_copy(dst_h.at[pl.ds(sid * TR + AR, TR - AR)],
                                dst_v.at[pl.ds(0, TR - AR)])

            do_edges(jnp.where(sid < 15, (TR - AR) // 2, 0))
            plsc.subcore_barrier()

            @pl.when(sid < 15)
            def _():
                pltpu.sync_copy(acc_s.at[pl.ds(r0, RCH)],
                                out_h.at[pl.ds(r0, RCH)])

            @pl.when(sid == 15)
            def _():
                pltpu.sync_copy(acc_s.at[pl.ds(15 * RCH, RLAST)],
                                out_h.at[pl.ds(15 * RCH, RLAST)])

        @pl.when(cid == 0)
        def _():
            run(xs0_h, out0_h)

        @pl.when(cid == 1)
        def _():
            run(xs1_h, out1_h)

    return k(xs0, xs1, src2, dst2)


def _p0_tc(x, deg0, deg1):
    def body(x_ref, d0_ref, d1_ref, xs0_ref, xs1_ref, dinv_ref):
        dinv = lax.rsqrt(d0_ref[...] + d1_ref[...] - 1.0)
        xs = x_ref[...] * dinv
        xs0_ref[...] = xs[:, :HF]
        xs1_ref[...] = xs[:, HF:]
        dinv_ref[...] = dinv

    return pl.pallas_call(
        body,
        grid=(NBLK,),
        in_specs=[pl.BlockSpec((R, IN), lambda i: (i, 0)),
                  pl.BlockSpec((R, 1), lambda i: (i, 0)),
                  pl.BlockSpec((R, 1), lambda i: (i, 0))],
        out_specs=[pl.BlockSpec((R, HF), lambda i: (i, 0)),
                   pl.BlockSpec((R, HF), lambda i: (i, 0)),
                   pl.BlockSpec((R, 1), lambda i: (i, 0))],
        out_shape=[jax.ShapeDtypeStruct((N, HF), _f32),
                   jax.ShapeDtypeStruct((N, HF), _f32),
                   jax.ShapeDtypeStruct((N, 1), _f32)],
    )(x, deg0, deg1)


def _m12_tc(a0, a1, dinv, W1, b1, gamma1, beta1, W2):
    def body(a0_ref, a1_ref, dinv_ref, w1_ref, b1_ref, g_ref, bt_ref,
             w2_ref, y0_ref, y1_ref, h_s, s_s, q_s):
        ph = pl.program_id(0)
        i = pl.program_id(1)

        @pl.when(ph == 0)
        def _():
            z = (jnp.concatenate([a0_ref[...], a1_ref[...]], axis=1)
                 * dinv_ref[...])
            h = jnp.dot(z, w1_ref[...],
                        preferred_element_type=_f32) + b1_ref[...]
            h_s[pl.ds(i * R, R), :] = h

            @pl.when(i == 0)
            def _():
                s_s[...] = jnp.zeros_like(s_s)
                q_s[...] = jnp.zeros_like(q_s)

            s_s[...] += jnp.sum(h, axis=0, keepdims=True)
            q_s[...] += jnp.sum(h * h, axis=0, keepdims=True)

        @pl.when(ph == 1)
        def _():
            mu = s_s[...] * (1.0 / N)
            var = q_s[...] * (1.0 / N) - mu * mu
            a_ = g_ref[...] * lax.rsqrt(var + 1e-5)
            c_ = bt_ref[...] - mu * a_
            t = jnp.maximum(h_s[pl.ds(i * R, R), :] * a_ + c_, 0.0)
            y = (jnp.dot(t, w2_ref[...], preferred_element_type=_f32)
                 * dinv_ref[...])
            y0_ref[...] = y[:, :HF]
            y1_ref[...] = y[:, HF:]

    return pl.pallas_call(
        body,
        grid=(2, NBLK),
        in_specs=[pl.BlockSpec((R, HF), lambda ph, i: ((1 - ph) * i, 0)),
                  pl.BlockSpec((R, HF), lambda ph, i: ((1 - ph) * i, 0)),
                  pl.BlockSpec((R, 1), lambda ph, i: (i, 0)),
                  pl.BlockSpec((IN, HID), lambda ph, i: (0, 0)),
                  pl.BlockSpec((1, HID), lambda ph, i: (0, 0)),
                  pl.BlockSpec((1, HID), lambda ph, i: (0, 0)),
                  pl.BlockSpec((1, HID), lambda ph, i: (0, 0)),
                  pl.BlockSpec((HID, OUT), lambda ph, i: (0, 0))],
        out_specs=[pl.BlockSpec((R, HF), lambda ph, i: (ph * i, 0)),
                   pl.BlockSpec((R, HF), lambda ph, i: (ph * i, 0))],
        out_shape=[jax.ShapeDtypeStruct((N, HF), _f32),
                   jax.ShapeDtypeStruct((N, HF), _f32)],
        scratch_shapes=[pltpu.VMEM((N, HID), _f32),
                        pltpu.VMEM((1, HID), _f32),
                        pltpu.VMEM((1, HID), _f32)],
    )(a0, a1, dinv, W1, b1, gamma1, beta1, W2)


def _m34_tc(g0, g1, dinv, b2, gamma2, beta2, batch3):
    def body(g0_ref, g1_ref, dinv_ref, b_ref, g2_ref, bt2_ref, ids_ref,
             out_ref, v_s, s_s, q_s, pool_s, cnt_s):
        ph = pl.program_id(0)
        i = pl.program_id(1)

        @pl.when(ph == 0)
        def _():
            v = (jnp.concatenate([g0_ref[...], g1_ref[...]], axis=1)
                 * dinv_ref[...] + b_ref[...])
            v_s[pl.ds(i * R, R), :] = v

            @pl.when(i == 0)
            def _():
                s_s[...] = jnp.zeros_like(s_s)
                q_s[...] = jnp.zeros_like(q_s)

            s_s[...] += jnp.sum(v, axis=0, keepdims=True)
            q_s[...] += jnp.sum(v * v, axis=0, keepdims=True)

        @pl.when(ph == 1)
        def _():
            mu = s_s[...] * (1.0 / N)
            var = q_s[...] * (1.0 / N) - mu * mu
            a_ = g2_ref[...] * lax.rsqrt(var + 1e-5)
            c_ = bt2_ref[...] - mu * a_
            w = jnp.maximum(v_s[pl.ds(i * R, R), :] * a_ + c_, 0.0)
            ids = ids_ref[...].reshape(R, 1)
            oh = (ids == lax.broadcasted_iota(jnp.int32,
                                              (R, G), 1)).astype(_f32)

            @pl.when(i == 0)
            def _():
                pool_s[...] = jnp.zeros_like(pool_s)
                cnt_s[...] = jnp.zeros_like(cnt_s)

            pool_s[...] += lax.dot_general(
                oh, w, (((0,), (0,)), ((), ())), preferred_element_type=_f32)
            cnt_s[...] += jnp.sum(oh, axis=0, keepdims=True)

            @pl.when(i == NBLK - 1)
            def _():
                out_ref[...] = pool_s[...] / jnp.maximum(
                    cnt_s[...].reshape(G, 1), 1.0)

    return pl.pallas_call(
        body,
        grid=(2, NBLK),
        in_specs=[pl.BlockSpec((R, HF), lambda ph, i: ((1 - ph) * i, 0)),
                  pl.BlockSpec((R, HF), lambda ph, i: ((1 - ph) * i, 0)),
                  pl.BlockSpec((R, 1), lambda ph, i: ((1 - ph) * i, 0)),
                  pl.BlockSpec((1, OUT), lambda ph, i: (0, 0)),
                  pl.BlockSpec((1, OUT), lambda ph, i: (0, 0)),
                  pl.BlockSpec((1, OUT), lambda ph, i: (0, 0)),
                  pl.BlockSpec((1, 1, R), lambda ph, i: (ph * i, 0, 0))],
        out_specs=pl.BlockSpec((G, OUT), lambda ph, i: (0, 0)),
        out_shape=jax.ShapeDtypeStruct((G, OUT), _f32),
        scratch_shapes=[pltpu.VMEM((N, OUT), _f32),
                        pltpu.VMEM((1, OUT), _f32),
                        pltpu.VMEM((1, OUT), _f32),
                        pltpu.VMEM((G, OUT), _f32),
                        pltpu.VMEM((1, G), _f32)],
    )(g0, g1, dinv, b2, gamma2, beta2, batch3)


def kernel(x, edge_index, batch, W1, b1, gamma1, beta1, W2, b2, gamma2, beta2):
    src2 = edge_index[0].reshape(ER, EB)
    dst2 = edge_index[1].reshape(ER, EB)
    ones_n = jnp.ones((N,), _f32)

    deg0, deg1 = _deg_sc(ones_n, dst2)
    xs0, xs1, dinv = _p0_tc(x, deg0.reshape(N, 1), deg1.reshape(N, 1))
    a0, a1 = _agg_sc(xs0, xs1, src2, dst2)

    y0, y1 = _m12_tc(a0, a1, dinv, W1, b1.reshape(1, HID),
                     gamma1.reshape(1, HID), beta1.reshape(1, HID), W2)
    g0, g1 = _agg_sc(y0, y1, src2, dst2)

    return _m34_tc(g0, g1, dinv, b2.reshape(1, OUT),
                   gamma2.reshape(1, OUT), beta2.reshape(1, OUT),
                   batch.reshape(NBLK, 1, R))


# speedup vs baseline: 1.0785x; 1.0785x over previous
import functools

import jax
import jax.numpy as jnp
from jax import lax
from jax.experimental import pallas as pl
from jax.experimental.pallas import tpu as pltpu
from jax.experimental.pallas import tpu_sc as plsc

N = 10000
E = 160000
IN = 256
HID = 512
OUT = 256
G = 64

NS = 16
RCH = 632
RLAST = N - 15 * RCH
EB = 128
ER = E // EB
TR = 80
AR = 64
HB = EB // 2
DR = 40

R = 1000
NBLK = N // R
HF = IN // 2

_f32 = jnp.float32


def _sc_mesh():
    return plsc.VectorSubcoreMesh(core_axis_name="c", subcore_axis_name="s")


def _deg_sc(ones_n, dst2):
    @functools.partial(
        pl.kernel,
        out_type=[jax.ShapeDtypeStruct((N,), _f32),
                  jax.ShapeDtypeStruct((N,), _f32)],
        mesh=_sc_mesh(),
        scratch_types=[
            pltpu.VMEM((DR, EB), jnp.int32),
            pltpu.VMEM((EB,), _f32),
            pltpu.VMEM((1000,), _f32),
            pltpu.VMEM_SHARED((N,), _f32),
        ],
    )
    def k(ones_h, dst_h, deg0_h, deg1_h, dst_v, one_v, tmp_v, deg_s):
        cid = lax.axis_index("c")
        sid = lax.axis_index("s")
        wid = cid * NS + sid

        @pl.when(wid < 31)
        def _():
            pltpu.sync_copy(dst_h.at[pl.ds(wid * DR, DR)], dst_v)

        @pl.when(wid == 31)
        def _():
            pltpu.sync_copy(dst_h.at[pl.ds(31 * DR, 10)],
                            dst_v.at[pl.ds(0, 10)])

        pltpu.sync_copy(ones_h.at[pl.ds(0, EB)], one_v)

        @pl.when(sid < 10)
        def _():
            pltpu.sync_copy(ones_h.at[pl.ds(sid * 1000, 1000)], tmp_v)
            pltpu.sync_copy(tmp_v, deg_s.at[pl.ds(sid * 1000, 1000)])

        plsc.subcore_barrier()
        nrow = jnp.where(wid < 31, DR, 10)

        def body(j, carry):
            pltpu.sync_copy(one_v, deg_s.at[dst_v.at[j]], add=True)
            return carry

        lax.fori_loop(0, nrow, body, 0)
        plsc.subcore_barrier()

        @pl.when(sid < 10)
        def _():
            pltpu.sync_copy(deg_s.at[pl.ds(sid * 1000, 1000)], tmp_v)

            @pl.when(cid == 0)
            def _():
                pltpu.sync_copy(tmp_v, deg0_h.at[pl.ds(sid * 1000, 1000)])

            @pl.when(cid == 1)
            def _():
                pltpu.sync_copy(tmp_v, deg1_h.at[pl.ds(sid * 1000, 1000)])

    return k(ones_n, dst2)


def _agg_sc(xs0, xs1, src2, dst2):
    @functools.partial(
        pl.kernel,
        out_type=[jax.ShapeDtypeStruct((N, HF), _f32),
                  jax.ShapeDtypeStruct((N, HF), _f32)],
        mesh=_sc_mesh(),
        scratch_types=[
            pltpu.VMEM((AR, EB), jnp.int32),
            pltpu.VMEM((AR, EB), jnp.int32),
            pltpu.VMEM((HB, HF), _f32),
            pltpu.VMEM((HB, HF), _f32),
            pltpu.VMEM((HB, HF), _f32),
            pltpu.VMEM((HB, HF), _f32),
            pltpu.VMEM_SHARED((N, HF), _f32),
            pltpu.SemaphoreType.DMA,
            pltpu.SemaphoreType.DMA,
            pltpu.SemaphoreType.DMA,
        ],
    )
    def k(xs0_h, xs1_h, src_h, dst_h, out0_h, out1_h,
          src_v, dst_v, rows_a, rows_b, rows_c, rows_d, acc_s,
          sem_g0, sem_s0, sem_s1):
        cid = lax.axis_index("c")
        sid = lax.axis_index("s")
        r0 = sid * RCH

        def run(xs_h, out_h):
            @pl.when(sid < 15)
            def _():
                pltpu.sync_copy(xs_h.at[pl.ds(r0, RCH)],
                                acc_s.at[pl.ds(r0, RCH)])

            @pl.when(sid == 15)
            def _():
                pltpu.sync_copy(xs_h.at[pl.ds(15 * RCH, RLAST)],
                                acc_s.at[pl.ds(15 * RCH, RLAST)])

            plsc.subcore_barrier()

            def gather(row, half, buf, sem):
                pltpu.async_copy(
                    xs_h.at[src_v.at[row, pl.ds(half * HB, HB)]], buf, sem)

            def gather_wait(row, half, buf, sem):
                pltpu.make_async_copy(
                    xs_h.at[src_v.at[row, pl.ds(half * HB, HB)]], buf,
                    sem).wait()

            def scat(row, half, buf, sem):
                pltpu.async_copy(
                    buf, acc_s.at[dst_v.at[row, pl.ds(half * HB, HB)]],
                    sem, add=True)

            def scat_wait(row, half, buf, sem):
                pltpu.make_async_copy(
                    buf, acc_s.at[dst_v.at[row, pl.ds(half * HB, HB)]],
                    sem).wait()

            def do_edges(npair):
                @pl.when(npair > 0)
                def _():
                    gather(0, 0, rows_a, sem_g0)
                    gather(0, 1, rows_b, sem_g0)

                    def body(t, carry):
                        r_e = 2 * t
                        r_o = 2 * t + 1
                        gather_wait(r_e, 0, rows_a, sem_g0)
                        gather_wait(r_e, 1, rows_b, sem_g0)
                        scat(r_e, 0, rows_a, sem_s0)
                        scat(r_e, 1, rows_b, sem_s0)

                        @pl.when(t > 0)
                        def _():
                            scat_wait(r_o - 2, 0, rows_c, sem_s1)
                            scat_wait(r_o - 2, 1, rows_d, sem_s1)

                        gather(r_o, 0, rows_c, sem_g0)
                        gather(r_o, 1, rows_d, sem_g0)
                        gather_wait(r_o, 0, rows_c, sem_g0)
                        gather_wait(r_o, 1, rows_d, sem_g0)
                        scat(r_o, 0, rows_c, sem_s1)
                        scat(r_o, 1, rows_d, sem_s1)
                        scat_wait(r_e, 0, rows_a, sem_s0)
                        scat_wait(r_e, 1, rows_b, sem_s0)

                        @pl.when(t + 1 < npair)
                        def _():
                            gather(r_e + 2, 0, rows_a, sem_g0)
                            gather(r_e + 2, 1, rows_b, sem_g0)

                        return carry

                    lax.fori_loop(0, npair, body, 0)
                    scat_wait(2 * npair - 1, 0, rows_c, sem_s1)
                    scat_wait(2 * npair - 1, 1, rows_d, sem_s1)

            @pl.when(sid < 15)
            def _():
                pltpu.sync_copy(src_h.at[pl.ds(sid * TR, AR)], src_v)
                pltpu.sync_copy(dst_h.at[pl.ds(sid * TR, AR)], dst_v)

            @pl.when(sid == 15)
            def _():
                pltpu.sync_copy(src_h.at[pl.ds(15 * TR, 48)],
                                src_v.at[pl.ds(0, 48)])
                pltpu.sync_copy(src_h.at[pl.ds(15 * TR + 48, 2)],
                                src_v.at[pl.ds(48, 2)])
                pltpu.sync_copy(dst_h.at[pl.ds(15 * TR, 48)],
                                dst_v.at[pl.ds(0, 48)])
                pltpu.sync_copy(dst_h.at[pl.ds(15 * TR + 48, 2)],
                                dst_v.at[pl.ds(48, 2)])

            do_edges(jnp.where(sid < 15, AR // 2, 25))

            @pl.when(sid < 15)
            def _():
                pltpu.sync_copy(src_h.at[pl.ds(sid * TR + AR, TR - AR)],
                                src_v.at[pl.ds(0, TR - AR)])
                pltpu.sync_copy(dst_h.at[pl.ds(sid * TR + AR, TR - AR)],
                                dst_v.at[pl.ds(0, TR - AR)])

            do_edges(jnp.where(sid < 15, (TR - AR) // 2, 0))
            plsc.subcore_barrier()

            @pl.when(sid < 15)
            def _():
                pltpu.sync_copy(acc_s.at[pl.ds(r0, RCH)],
                                out_h.at[pl.ds(r0, RCH)])

            @pl.when(sid == 15)
            def _():
                pltpu.sync_copy(acc_s.at[pl.ds(15 * RCH, RLAST)],
                                out_h.at[pl.ds(15 * RCH, RLAST)])

        @pl.when(cid == 0)
        def _():
            run(xs0_h, out0_h)

        @pl.when(cid == 1)
        def _():
            run(xs1_h, out1_h)

    return k(xs0, xs1, src2, dst2)


def _p0_tc(x, deg0, deg1):
    def body(x_ref, d0_ref, d1_ref, xs0_ref, xs1_ref, dinv_ref):
        dinv = lax.rsqrt(d0_ref[...] + d1_ref[...] - 1.0)
        xs = x_ref[...] * dinv
        xs0_ref[...] = xs[:, :HF]
        xs1_ref[...] = xs[:, HF:]
        dinv_ref[...] = dinv

    return pl.pallas_call(
        body,
        grid=(NBLK,),
        in_specs=[pl.BlockSpec((R, IN), lambda i: (i, 0)),
                  pl.BlockSpec((R, 1), lambda i: (i, 0)),
                  pl.BlockSpec((R, 1), lambda i: (i, 0))],
        out_specs=[pl.BlockSpec((R, HF), lambda i: (i, 0)),
                   pl.BlockSpec((R, HF), lambda i: (i, 0)),
                   pl.BlockSpec((R, 1), lambda i: (i, 0))],
        out_shape=[jax.ShapeDtypeStruct((N, HF), _f32),
                   jax.ShapeDtypeStruct((N, HF), _f32),
                   jax.ShapeDtypeStruct((N, 1), _f32)],
    )(x, deg0, deg1)


def _m12_tc(a0, a1, dinv, W1, b1, gamma1, beta1, W2):
    def body(a0_ref, a1_ref, dinv_ref, w1_ref, b1_ref, g_ref, bt_ref,
             w2_ref, y0_ref, y1_ref, h_s, s_s, q_s):
        ph = pl.program_id(0)
        i = pl.program_id(1)

        @pl.when(ph == 0)
        def _():
            z = (jnp.concatenate([a0_ref[...], a1_ref[...]], axis=1)
                 * dinv_ref[...])
            h = jnp.dot(z, w1_ref[...],
                        preferred_element_type=_f32) + b1_ref[...]
            h_s[pl.ds(i * R, R), :] = h

            @pl.when(i == 0)
            def _():
                s_s[...] = jnp.zeros_like(s_s)
                q_s[...] = jnp.zeros_like(q_s)

            s_s[...] += jnp.sum(h, axis=0, keepdims=True)
            q_s[...] += jnp.sum(h * h, axis=0, keepdims=True)

        @pl.when(ph == 1)
        def _():
            mu = s_s[...] * (1.0 / N)
            var = q_s[...] * (1.0 / N) - mu * mu
            a_ = g_ref[...] * lax.rsqrt(var + 1e-5)
            c_ = bt_ref[...] - mu * a_
            t = jnp.maximum(h_s[pl.ds(i * R, R), :] * a_ + c_, 0.0)
            y = (jnp.dot(t, w2_ref[...], preferred_element_type=_f32)
                 * dinv_ref[...])
            y0_ref[...] = y[:, :HF]
            y1_ref[...] = y[:, HF:]

    return pl.pallas_call(
        body,
        grid=(2, NBLK),
        in_specs=[pl.BlockSpec((R, HF), lambda ph, i: ((1 - ph) * i, 0)),
                  pl.BlockSpec((R, HF), lambda ph, i: ((1 - ph) * i, 0)),
                  pl.BlockSpec((R, 1), lambda ph, i: (i, 0)),
                  pl.BlockSpec((IN, HID), lambda ph, i: (0, 0)),
                  pl.BlockSpec((1, HID), lambda ph, i: (0, 0)),
                  pl.BlockSpec((1, HID), lambda ph, i: (0, 0)),
                  pl.BlockSpec((1, HID), lambda ph, i: (0, 0)),
                  pl.BlockSpec((HID, OUT), lambda ph, i: (0, 0))],
        out_specs=[pl.BlockSpec((R, HF), lambda ph, i: (ph * i, 0)),
                   pl.BlockSpec((R, HF), lambda ph, i: (ph * i, 0))],
        out_shape=[jax.ShapeDtypeStruct((N, HF), _f32),
                   jax.ShapeDtypeStruct((N, HF), _f32)],
        scratch_shapes=[pltpu.VMEM((N, HID), _f32),
                        pltpu.VMEM((1, HID), _f32),
                        pltpu.VMEM((1, HID), _f32)],
    )(a0, a1, dinv, W1, b1, gamma1, beta1, W2)


def _m34_tc(g0, g1, dinv, b2, gamma2, beta2, batch3):
    def body(g0_ref, g1_ref, dinv_ref, b_ref, g2_ref, bt2_ref, ids_ref,
             out_ref, v_s, s_s, q_s, pool_s, cnt_s):
        ph = pl.program_id(0)
        i = pl.program_id(1)

        @pl.when(ph == 0)
        def _():
            v = (jnp.concatenate([g0_ref[...], g1_ref[...]], axis=1)
                 * dinv_ref[...] + b_ref[...])
            v_s[pl.ds(i * R, R), :] = v

            @pl.when(i == 0)
            def _():
                s_s[...] = jnp.zeros_like(s_s)
                q_s[...] = jnp.zeros_like(q_s)

            s_s[...] += jnp.sum(v, axis=0, keepdims=True)
            q_s[...] += jnp.sum(v * v, axis=0, keepdims=True)

        @pl.when(ph == 1)
        def _():
            mu = s_s[...] * (1.0 / N)
            var = q_s[...] * (1.0 / N) - mu * mu
            a_ = g2_ref[...] * lax.rsqrt(var + 1e-5)
            c_ = bt2_ref[...] - mu * a_
            w = jnp.maximum(v_s[pl.ds(i * R, R), :] * a_ + c_, 0.0)
            ids = ids_ref[...].reshape(R, 1)
            oh = (ids == lax.broadcasted_iota(jnp.int32,
                                              (R, G), 1)).astype(_f32)

            @pl.when(i == 0)
            def _():
                pool_s[...] = jnp.zeros_like(pool_s)
                cnt_s[...] = jnp.zeros_like(cnt_s)

            pool_s[...] += lax.dot_general(
                oh, w, (((0,), (0,)), ((), ())), preferred_element_type=_f32)
            cnt_s[...] += jnp.sum(oh, axis=0, keepdims=True)

            @pl.when(i == NBLK - 1)
            def _():
                out_ref[...] = pool_s[...] / jnp.maximum(
                    cnt_s[...].reshape(G, 1), 1.0)

    return pl.pallas_call(
        body,
        grid=(2, NBLK),
        in_specs=[pl.BlockSpec((R, HF), lambda ph, i: ((1 - ph) * i, 0)),
                  pl.BlockSpec((R, HF), lambda ph, i: ((1 - ph) * i, 0)),
                  pl.BlockSpec((R, 1), lambda ph, i: ((1 - ph) * i, 0)),
                  pl.BlockSpec((1, OUT), lambda ph, i: (0, 0)),
                  pl.BlockSpec((1, OUT), lambda ph, i: (0, 0)),
                  pl.BlockSpec((1, OUT), lambda ph, i: (0, 0)),
                  pl.BlockSpec((1, 1, R), lambda ph, i: (ph * i, 0, 0))],
        out_specs=pl.BlockSpec((G, OUT), lambda ph, i: (0, 0)),
        out_shape=jax.ShapeDtypeStruct((G, OUT), _f32),
        scratch_shapes=[pltpu.VMEM((N, OUT), _f32),
                        pltpu.VMEM((1, OUT), _f32),
                        pltpu.VMEM((1, OUT), _f32),
                        pltpu.VMEM((G, OUT), _f32),
                        pltpu.VMEM((1, G), _f32)],
    )(g0, g1, dinv, b2, gamma2, beta2, batch3)


def kernel(x, edge_index, batch, W1, b1, gamma1, beta1, W2, b2, gamma2, beta2):
    src2 = edge_index[0].reshape(ER, EB)
    dst2 = edge_index[1].reshape(ER, EB)
    ones_n = jnp.ones((N,), _f32)

    deg0, deg1 = _deg_sc(ones_n, dst2)
    xs0, xs1, dinv = _p0_tc(x, deg0.reshape(N, 1), deg1.reshape(N, 1))
    a0, a1 = _agg_sc(xs0, xs1, src2, dst2)

    y0, y1 = _m12_tc(a0, a1, dinv, W1, b1.reshape(1, HID),
                     gamma1.reshape(1, HID), beta1.reshape(1, HID), W2)
    g0, g1 = _agg_sc(y0, y1, src2, dst2)

    return _m34_tc(g0, g1, dinv, b2.reshape(1, OUT),
                   gamma2.reshape(1, OUT), beta2.reshape(1, OUT),
                   batch.reshape(NBLK, 1, R))


# speedup vs baseline: 1.0822x; 1.0034x over previous
import functools

import jax
import jax.numpy as jnp
from jax import lax
from jax.experimental import pallas as pl
from jax.experimental.pallas import tpu as pltpu
from jax.experimental.pallas import tpu_sc as plsc

N = 10000
E = 160000
IN = 256
HID = 512
OUT = 256
G = 64

NS = 16
RCH = 632
RLAST = N - 15 * RCH
EB = 128
ER = E // EB
TR = 80
AR = 64
HB = EB // 2
DR = 40

R = 1000
NBLK = N // R
HF = IN // 2

_f32 = jnp.float32


def _sc_mesh():
    return plsc.VectorSubcoreMesh(core_axis_name="c", subcore_axis_name="s")


def _deg_sc(ones_n, dst2):
    @functools.partial(
        pl.kernel,
        out_type=[jax.ShapeDtypeStruct((N,), _f32),
                  jax.ShapeDtypeStruct((N,), _f32)],
        mesh=_sc_mesh(),
        scratch_types=[
            pltpu.VMEM((DR, EB), jnp.int32),
            pltpu.VMEM((EB,), _f32),
            pltpu.VMEM((1000,), _f32),
            pltpu.VMEM_SHARED((N,), _f32),
        ],
    )
    def k(ones_h, dst_h, deg0_h, deg1_h, dst_v, one_v, tmp_v, deg_s):
        cid = lax.axis_index("c")
        sid = lax.axis_index("s")
        wid = cid * NS + sid

        @pl.when(wid < 31)
        def _():
            pltpu.sync_copy(dst_h.at[pl.ds(wid * DR, DR)], dst_v)

        @pl.when(wid == 31)
        def _():
            pltpu.sync_copy(dst_h.at[pl.ds(31 * DR, 10)],
                            dst_v.at[pl.ds(0, 10)])

        pltpu.sync_copy(ones_h.at[pl.ds(0, EB)], one_v)

        @pl.when(sid < 10)
        def _():
            pltpu.sync_copy(ones_h.at[pl.ds(sid * 1000, 1000)], tmp_v)
            pltpu.sync_copy(tmp_v, deg_s.at[pl.ds(sid * 1000, 1000)])

        plsc.subcore_barrier()
        nrow = jnp.where(wid < 31, DR, 10)

        def body(j, carry):
            pltpu.sync_copy(one_v, deg_s.at[dst_v.at[j]], add=True)
            return carry

        lax.fori_loop(0, nrow, body, 0)
        plsc.subcore_barrier()

        @pl.when(sid < 10)
        def _():
            pltpu.sync_copy(deg_s.at[pl.ds(sid * 1000, 1000)], tmp_v)

            @pl.when(cid == 0)
            def _():
                pltpu.sync_copy(tmp_v, deg0_h.at[pl.ds(sid * 1000, 1000)])

            @pl.when(cid == 1)
            def _():
                pltpu.sync_copy(tmp_v, deg1_h.at[pl.ds(sid * 1000, 1000)])

    return k(ones_n, dst2)


def _agg_sc(xs0, xs1, src2, dst2):
    @functools.partial(
        pl.kernel,
        out_type=[jax.ShapeDtypeStruct((N, HF), _f32),
                  jax.ShapeDtypeStruct((N, HF), _f32)],
        mesh=_sc_mesh(),
        scratch_types=[
            pltpu.VMEM((AR, EB), jnp.int32),
            pltpu.VMEM((AR, EB), jnp.int32),
            pltpu.VMEM((EB, HF), _f32),
            pltpu.VMEM((EB, HF), _f32),
            pltpu.VMEM_SHARED((N, HF), _f32),
            pltpu.SemaphoreType.DMA,
            pltpu.SemaphoreType.DMA,
            pltpu.SemaphoreType.DMA,
        ],
    )
    def k(xs0_h, xs1_h, src_h, dst_h, out0_h, out1_h,
          src_v, dst_v, rows_a, rows_b, acc_s,
          sem_g0, sem_s0, sem_s1):
        cid = lax.axis_index("c")
        sid = lax.axis_index("s")
        r0 = sid * RCH

        def run(xs_h, out_h):
            @pl.when(sid < 15)
            def _():
                pltpu.sync_copy(xs_h.at[pl.ds(r0, RCH)],
                                acc_s.at[pl.ds(r0, RCH)])

            @pl.when(sid == 15)
            def _():
                pltpu.sync_copy(xs_h.at[pl.ds(15 * RCH, RLAST)],
                                acc_s.at[pl.ds(15 * RCH, RLAST)])

            plsc.subcore_barrier()

            def gather(row, buf):
                pltpu.async_copy(xs_h.at[src_v.at[row]], buf, sem_g0)

            def gather_wait(row, buf):
                pltpu.make_async_copy(xs_h.at[src_v.at[row]], buf,
                                      sem_g0).wait()

            def scat(row, buf, sem):
                pltpu.async_copy(buf, acc_s.at[dst_v.at[row]], sem, add=True)

            def scat_wait(row, buf, sem):
                pltpu.make_async_copy(buf, acc_s.at[dst_v.at[row]],
                                      sem).wait()

            def do_edges(npair):
                @pl.when(npair > 0)
                def _():
                    gather(0, rows_a)

                    def body(t, carry):
                        r_e = 2 * t
                        r_o = 2 * t + 1
                        gather_wait(r_e, rows_a)

                        @pl.when(t > 0)
                        def _():
                            scat_wait(r_o - 2, rows_b, sem_s1)

                        gather(r_o, rows_b)
                        scat(r_e, rows_a, sem_s0)
                        gather_wait(r_o, rows_b)
                        scat_wait(r_e, rows_a, sem_s0)

                        @pl.when(t + 1 < npair)
                        def _():
                            gather(r_e + 2, rows_a)

                        scat(r_o, rows_b, sem_s1)
                        return carry

                    lax.fori_loop(0, npair, body, 0)
                    scat_wait(2 * npair - 1, rows_b, sem_s1)

            @pl.when(sid < 15)
            def _():
                pltpu.sync_copy(src_h.at[pl.ds(sid * TR, AR)], src_v)
                pltpu.sync_copy(dst_h.at[pl.ds(sid * TR, AR)], dst_v)

            @pl.when(sid == 15)
            def _():
                pltpu.sync_copy(src_h.at[pl.ds(15 * TR, 48)],
                                src_v.at[pl.ds(0, 48)])
                pltpu.sync_copy(src_h.at[pl.ds(15 * TR + 48, 2)],
                                src_v.at[pl.ds(48, 2)])
                pltpu.sync_copy(dst_h.at[pl.ds(15 * TR, 48)],
                                dst_v.at[pl.ds(0, 48)])
                pltpu.sync_copy(dst_h.at[pl.ds(15 * TR + 48, 2)],
                                dst_v.at[pl.ds(48, 2)])

            do_edges(jnp.where(sid < 15, AR // 2, 25))

            @pl.when(sid < 15)
            def _():
                pltpu.sync_copy(src_h.at[pl.ds(sid * TR + AR, TR - AR)],
                                src_v.at[pl.ds(0, TR - AR)])
                pltpu.sync_copy(dst_h.at[pl.ds(sid * TR + AR, TR - AR)],
                                dst_v.at[pl.ds(0, TR - AR)])

            do_edges(jnp.where(sid < 15, (TR - AR) // 2, 0))
            plsc.subcore_barrier()

            @pl.when(sid < 15)
            def _():
                pltpu.sync_copy(acc_s.at[pl.ds(r0, RCH)],
                                out_h.at[pl.ds(r0, RCH)])

            @pl.when(sid == 15)
            def _():
                pltpu.sync_copy(acc_s.at[pl.ds(15 * RCH, RLAST)],
                                out_h.at[pl.ds(15 * RCH, RLAST)])

        @pl.when(cid == 0)
        def _():
            run(xs0_h, out0_h)

        @pl.when(cid == 1)
        def _():
            run(xs1_h, out1_h)

    return k(xs0, xs1, src2, dst2)


def _p0_tc(x, deg0, deg1):
    def body(x_ref, d0_ref, d1_ref, xs0_ref, xs1_ref, dinv_ref):
        dinv = lax.rsqrt(d0_ref[...] + d1_ref[...] - 1.0)
        xs = x_ref[...] * dinv
        xs0_ref[...] = xs[:, :HF]
        xs1_ref[...] = xs[:, HF:]
        dinv_ref[...] = dinv

    return pl.pallas_call(
        body,
        grid=(NBLK,),
        in_specs=[pl.BlockSpec((R, IN), lambda i: (i, 0)),
                  pl.BlockSpec((R, 1), lambda i: (i, 0)),
                  pl.BlockSpec((R, 1), lambda i: (i, 0))],
        out_specs=[pl.BlockSpec((R, HF), lambda i: (i, 0)),
                   pl.BlockSpec((R, HF), lambda i: (i, 0)),
                   pl.BlockSpec((R, 1), lambda i: (i, 0))],
        out_shape=[jax.ShapeDtypeStruct((N, HF), _f32),
                   jax.ShapeDtypeStruct((N, HF), _f32),
                   jax.ShapeDtypeStruct((N, 1), _f32)],
    )(x, deg0, deg1)


def _m12_tc(a0, a1, dinv, W1, b1, gamma1, beta1, W2):
    def body(a0_ref, a1_ref, dinv_ref, w1_ref, b1_ref, g_ref, bt_ref,
             w2_ref, y0_ref, y1_ref, h_s, s_s, q_s):
        ph = pl.program_id(0)
        i = pl.program_id(1)

        @pl.when(ph == 0)
        def _():
            z = (jnp.concatenate([a0_ref[...], a1_ref[...]], axis=1)
                 * dinv_ref[...])
            h = jnp.dot(z, w1_ref[...],
                        preferred_element_type=_f32) + b1_ref[...]
            h_s[pl.ds(i * R, R), :] = h

            @pl.when(i == 0)
            def _():
                s_s[...] = jnp.zeros_like(s_s)
                q_s[...] = jnp.zeros_like(q_s)

            s_s[...] += jnp.sum(h, axis=0, keepdims=True)
            q_s[...] += jnp.sum(h * h, axis=0, keepdims=True)

        @pl.when(ph == 1)
        def _():
            mu = s_s[...] * (1.0 / N)
            var = q_s[...] * (1.0 / N) - mu * mu
            a_ = g_ref[...] * lax.rsqrt(var + 1e-5)
            c_ = bt_ref[...] - mu * a_
            t = jnp.maximum(h_s[pl.ds(i * R, R), :] * a_ + c_, 0.0)
            y = (jnp.dot(t, w2_ref[...], preferred_element_type=_f32)
                 * dinv_ref[...])
            y0_ref[...] = y[:, :HF]
            y1_ref[...] = y[:, HF:]

    return pl.pallas_call(
        body,
        grid=(2, NBLK),
        in_specs=[pl.BlockSpec((R, HF), lambda ph, i: ((1 - ph) * i, 0)),
                  pl.BlockSpec((R, HF), lambda ph, i: ((1 - ph) * i, 0)),
                  pl.BlockSpec((R, 1), lambda ph, i: (i, 0)),
                  pl.BlockSpec((IN, HID), lambda ph, i: (0, 0)),
                  pl.BlockSpec((1, HID), lambda ph, i: (0, 0)),
                  pl.BlockSpec((1, HID), lambda ph, i: (0, 0)),
                  pl.BlockSpec((1, HID), lambda ph, i: (0, 0)),
                  pl.BlockSpec((HID, OUT), lambda ph, i: (0, 0))],
        out_specs=[pl.BlockSpec((R, HF), lambda ph, i: (ph * i, 0)),
                   pl.BlockSpec((R, HF), lambda ph, i: (ph * i, 0))],
        out_shape=[jax.ShapeDtypeStruct((N, HF), _f32),
                   jax.ShapeDtypeStruct((N, HF), _f32)],
        scratch_shapes=[pltpu.VMEM((N, HID), _f32),
                        pltpu.VMEM((1, HID), _f32),
                        pltpu.VMEM((1, HID), _f32)],
    )(a0, a1, dinv, W1, b1, gamma1, beta1, W2)


def _m34_tc(g0, g1, dinv, b2, gamma2, beta2, batch3):
    def body(g0_ref, g1_ref, dinv_ref, b_ref, g2_ref, bt2_ref, ids_ref,
             out_ref, v_s, s_s, q_s, pool_s, cnt_s):
        ph = pl.program_id(0)
        i = pl.program_id(1)

        @pl.when(ph == 0)
        def _():
            v = (jnp.concatenate([g0_ref[...], g1_ref[...]], axis=1)
                 * dinv_ref[...] + b_ref[...])
            v_s[pl.ds(i * R, R), :] = v

            @pl.when(i == 0)
            def _():
                s_s[...] = jnp.zeros_like(s_s)
                q_s[...] = jnp.zeros_like(q_s)

            s_s[...] += jnp.sum(v, axis=0, keepdims=True)
            q_s[...] += jnp.sum(v * v, axis=0, keepdims=True)

        @pl.when(ph == 1)
        def _():
            mu = s_s[...] * (1.0 / N)
            var = q_s[...] * (1.0 / N) - mu * mu
            a_ = g2_ref[...] * lax.rsqrt(var + 1e-5)
            c_ = bt2_ref[...] - mu * a_
            w = jnp.maximum(v_s[pl.ds(i * R, R), :] * a_ + c_, 0.0)
            ids = ids_ref[...].reshape(R, 1)
            oh = (ids == lax.broadcasted_iota(jnp.int32,
                                              (R, G), 1)).astype(_f32)

            @pl.when(i == 0)
            def _():
                pool_s[...] = jnp.zeros_like(pool_s)
                cnt_s[...] = jnp.zeros_like(cnt_s)

            pool_s[...] += lax.dot_general(
                oh, w, (((0,), (0,)), ((), ())), preferred_element_type=_f32)
            cnt_s[...] += jnp.sum(oh, axis=0, keepdims=True)

            @pl.when(i == NBLK - 1)
            def _():
                out_ref[...] = pool_s[...] / jnp.maximum(
                    cnt_s[...].reshape(G, 1), 1.0)

    return pl.pallas_call(
        body,
        grid=(2, NBLK),
        in_specs=[pl.BlockSpec((R, HF), lambda ph, i: ((1 - ph) * i, 0)),
                  pl.BlockSpec((R, HF), lambda ph, i: ((1 - ph) * i, 0)),
                  pl.BlockSpec((R, 1), lambda ph, i: ((1 - ph) * i, 0)),
                  pl.BlockSpec((1, OUT), lambda ph, i: (0, 0)),
                  pl.BlockSpec((1, OUT), lambda ph, i: (0, 0)),
                  pl.BlockSpec((1, OUT), lambda ph, i: (0, 0)),
                  pl.BlockSpec((1, 1, R), lambda ph, i: (ph * i, 0, 0))],
        out_specs=pl.BlockSpec((G, OUT), lambda ph, i: (0, 0)),
        out_shape=jax.ShapeDtypeStruct((G, OUT), _f32),
        scratch_shapes=[pltpu.VMEM((N, OUT), _f32),
                        pltpu.VMEM((1, OUT), _f32),
                        pltpu.VMEM((1, OUT), _f32),
                        pltpu.VMEM((G, OUT), _f32),
                        pltpu.VMEM((1, G), _f32)],
    )(g0, g1, dinv, b2, gamma2, beta2, batch3)


def kernel(x, edge_index, batch, W1, b1, gamma1, beta1, W2, b2, gamma2, beta2):
    src2 = edge_index[0].reshape(ER, EB)
    dst2 = edge_index[1].reshape(ER, EB)
    ones_n = jnp.ones((N,), _f32)

    deg0, deg1 = _deg_sc(ones_n, dst2)
    xs0, xs1, dinv = _p0_tc(x, deg0.reshape(N, 1), deg1.reshape(N, 1))
    a0, a1 = _agg_sc(xs0, xs1, src2, dst2)

    y0, y1 = _m12_tc(a0, a1, dinv, W1, b1.reshape(1, HID),
                     gamma1.reshape(1, HID), beta1.reshape(1, HID), W2)
    g0, g1 = _agg_sc(y0, y1, src2, dst2)

    return _m34_tc(g0, g1, dinv, b2.reshape(1, OUT),
                   gamma2.reshape(1, OUT), beta2.reshape(1, OUT),
                   batch.reshape(NBLK, 1, R))
